# baseline (speedup 1.0000x reference)
import functools

import jax
import jax.numpy as jnp
from jax import lax
from jax.experimental import pallas as pl
from jax.experimental.pallas import tpu as pltpu
from jax.experimental.pallas import tpu_sc as plsc

_N = 50000
_B = 256
_E = 800000
_NPAD = 50176
_NTILES = 16
_C = 2000
_G = 400
_ZR = 98
_GB = 512


def _sc_segment_sum(x_halves, src, dst):
    npad, fp = x_halves[0].shape
    n_halves = len(x_halves)
    rpr = npad // 2
    tile_rows = rpr // _NTILES
    e_per_tile = _E // _NTILES
    n_chunks = e_per_tile // _C
    nb = _C // _G

    mesh = plsc.VectorSubcoreMesh(core_axis_name="c", subcore_axis_name="s")

    @functools.partial(
        pl.kernel,
        out_type=[jax.ShapeDtypeStruct((npad, fp), jnp.float32)
                  for _ in range(n_halves)],
        mesh=mesh,
        compiler_params=pltpu.CompilerParams(use_tc_tiling_on_sc=False),
        scratch_types=[
            pltpu.VMEM((_C,), jnp.int32),
            pltpu.VMEM((_C,), jnp.int32),
            pltpu.VMEM((2, 1, _G), jnp.int32),
            pltpu.VMEM((2, 1, _G), jnp.int32),
            pltpu.VMEM((2, _G, fp), jnp.float32),
            pltpu.VMEM((_ZR, fp), jnp.float32),
            pltpu.VMEM_SHARED((rpr + _GB, fp), jnp.float32),
            pltpu.SemaphoreType.DMA,
            pltpu.SemaphoreType.DMA,
            pltpu.SemaphoreType.DMA,
            pltpu.SemaphoreType.DMA,
        ],
    )
    def kern(*refs):
        x_hbms = refs[:n_halves]
        src_hbm, dst_hbm = refs[n_halves], refs[n_halves + 1]
        out_hbms = refs[n_halves + 2:2 * n_halves + 2]
        (dstb, srcb, sidx, sgix, stag, zbuf, acc,
         gsem0, gsem1, ssem0, ssem1) = refs[2 * n_halves + 2:]
        gsems, ssems = (gsem0, gsem1), (ssem0, ssem1)
        cid = lax.axis_index("c")
        sid = lax.axis_index("s")
        lo = cid * rpr

        def zrow(i, _):
            def zcol(j, _):
                zbuf[i, pl.ds(j * 16, 16)] = jnp.zeros((16,), jnp.float32)
                return 0
            return lax.fori_loop(0, fp // 16, zcol, 0)
        lax.fori_loop(0, _ZR, zrow, 0)

        for h in range(n_halves):
            x_hbm, out_hbm = x_hbms[h], out_hbms[h]
            for zi in range(tile_rows // _ZR):
                pltpu.sync_copy(
                    zbuf, acc.at[pl.ds(sid * tile_rows + zi * _ZR, _ZR)])
            plsc.subcore_barrier()

            e0 = sid * e_per_tile

            def chunk_body(ci, _):
                base = e0 + ci * _C
                pltpu.sync_copy(dst_hbm.at[pl.ds(base, _C)], dstb)
                pltpu.sync_copy(src_hbm.at[pl.ds(base, _C)], srcb)

                def mapf_mk(bi, p):
                    def mapf(j, _):
                        d = dstb[pl.ds(bi * _G + j * 16, 16)]
                        s = srcb[pl.ds(bi * _G + j * 16, 16)]
                        inr = (d >= lo) & (d < lo + rpr)
                        loc = jnp.where(inr, d - lo,
                                        rpr + (d & (_GB - 1)))
                        sidx[p, 0, pl.ds(j * 16, 16)] = loc
                        sgix[p, 0, pl.ds(j * 16, 16)] = s
                        return 0
                    return mapf

                gd = [None, None]
                sd = [None, None]
                for b in range(nb):
                    p = b % 2
                    q = 1 - p
                    if b >= 2:
                        sd[p].wait()
                    if b >= 1:
                        gd[q].wait()
                        sd[q] = pltpu.async_copy(
                            stag.at[q], acc.at[sidx.at[q, 0]], ssems[q],
                            add=True)
                    lax.fori_loop(0, _G // 16, mapf_mk(b, p), 0)
                    gd[p] = pltpu.async_copy(
                        x_hbm.at[sgix.at[p, 0]], stag.at[p], gsems[p])
                p = (nb - 1) % 2
                gd[p].wait()
                sd[p] = pltpu.async_copy(
                    stag.at[p], acc.at[sidx.at[p, 0]], ssems[p], add=True)
                sd[1 - p].wait()
                sd[p].wait()
                return 0

            lax.fori_loop(0, n_chunks, chunk_body, 0)
            plsc.subcore_barrier()
            pltpu.sync_copy(
                acc.at[pl.ds(sid * tile_rows, tile_rows)],
                out_hbm.at[pl.ds(lo + sid * tile_rows, tile_rows)])
            plsc.subcore_barrier()

    return kern(*x_halves, src, dst)


def _tc_gin(x, agg, wp, bp):
    npad, fp = x.shape
    blk = 512

    def body(x_ref, a_ref, w_ref, b_ref, o_ref):
        t = x_ref[...] + a_ref[...]
        y = jnp.dot(t, w_ref[...], preferred_element_type=jnp.float32)
        o_ref[...] = jnp.maximum(y + b_ref[...], 0.0)

    return pl.pallas_call(
        body,
        grid=(npad // blk,),
        in_specs=[
            pl.BlockSpec((blk, fp), lambda i: (i, 0)),
            pl.BlockSpec((blk, fp), lambda i: (i, 0)),
            pl.BlockSpec(wp.shape, lambda i: (0, 0)),
            pl.BlockSpec(bp.shape, lambda i: (0, 0)),
        ],
        out_specs=pl.BlockSpec((blk, fp), lambda i: (i, 0)),
        out_shape=jax.ShapeDtypeStruct((npad, fp), jnp.float32),
    )(x, agg, wp, bp)


def _tc_gin2_pool(h1, agg, wp, bp, bcol, bnext):
    npad, fp = h1.shape
    f2 = wp.shape[1]
    blk = 256
    nblk = npad // blk

    def body(h_ref, a_ref, w_ref, b_ref, bc_ref, bn_ref,
             pmax_ref, psum_ref, pcnt_ref):
        i = pl.program_id(0)

        @pl.when(i == 0)
        def _():
            pmax_ref[...] = jnp.zeros_like(pmax_ref)
            psum_ref[...] = jnp.zeros_like(psum_ref)
            pcnt_ref[...] = jnp.zeros_like(pcnt_ref)

        t = h_ref[...] + a_ref[...]
        y = jnp.dot(t, w_ref[...], preferred_element_type=jnp.float32)
        h2 = jnp.maximum(y + b_ref[...], 0.0)

        bc = bc_ref[...]
        bn = bn_ref[...]
        segs = lax.broadcasted_iota(jnp.int32, (1, _B), 1).astype(jnp.float32)
        m = (bc == segs).astype(jnp.float32)

        dn = (((0,), (0,)), ((), ()))
        psum_ref[...] += lax.dot_general(
            m, h2, dn, preferred_element_type=jnp.float32, precision=lax.Precision.HIGHEST)
        pcnt_ref[...] += lax.dot_general(
            m, jnp.ones((blk, 1), jnp.float32), dn,
            preferred_element_type=jnp.float32, precision=lax.Precision.HIGHEST)

        c = h2
        k = 1
        while k < blk:
            sb = jnp.concatenate(
                [jnp.full((k, 1), -1.0, jnp.float32), bc[:-k]], axis=0)
            cs = jnp.concatenate(
                [jnp.zeros((k, f2), jnp.float32), c[:-k]], axis=0)
            c = jnp.maximum(c, jnp.where(sb == bc, cs, 0.0))
            k *= 2
        row_last = lax.broadcasted_iota(jnp.int32, (blk, 1), 0) == (blk - 1)
        end = jnp.logical_or(bc != bn, row_last).astype(jnp.float32)
        placed = lax.dot_general(
            m * end, c, dn, preferred_element_type=jnp.float32, precision=lax.Precision.HIGHEST)
        pmax_ref[...] = jnp.maximum(pmax_ref[...], placed)

    return pl.pallas_call(
        body,
        grid=(nblk,),
        in_specs=[
            pl.BlockSpec((blk, fp), lambda i: (i, 0)),
            pl.BlockSpec((blk, fp), lambda i: (i, 0)),
            pl.BlockSpec(wp.shape, lambda i: (0, 0)),
            pl.BlockSpec(bp.shape, lambda i: (0, 0)),
            pl.BlockSpec((blk, 1), lambda i: (i, 0)),
            pl.BlockSpec((blk, 1), lambda i: (i, 0)),
        ],
        out_specs=[
            pl.BlockSpec((_B, f2), lambda i: (0, 0)),
            pl.BlockSpec((_B, f2), lambda i: (0, 0)),
            pl.BlockSpec((_B, 1), lambda i: (0, 0)),
        ],
        out_shape=[
            jax.ShapeDtypeStruct((_B, f2), jnp.float32),
            jax.ShapeDtypeStruct((_B, f2), jnp.float32),
            jax.ShapeDtypeStruct((_B, 1), jnp.float32),
        ],
    )(h1, agg, wp, bp, bcol, bnext)


def _tc_head(pmax, psum, pcnt, wg1a, wg1b, bg1, wg2, bg2, wf1, bf1, wf2, bf2):
    def body(pmax_ref, psum_ref, pcnt_ref, wg1a_ref, wg1b_ref, bg1_ref,
             wg2_ref, bg2_ref, wf1_ref, bf1_ref, wf2_ref, bf2_ref,
             xg_ref, z_ref):
        den = jnp.maximum(pcnt_ref[...], 1.0)
        mean = psum_ref[...] / den
        g1 = (jnp.dot(pmax_ref[...], wg1a_ref[...],
                      preferred_element_type=jnp.float32)
              + jnp.dot(mean, wg1b_ref[...],
                        preferred_element_type=jnp.float32)
              + bg1_ref[...])
        g1 = jnp.maximum(g1, 0.0)
        xg = jnp.dot(g1, wg2_ref[...],
                     preferred_element_type=jnp.float32) + bg2_ref[...]
        xg_ref[...] = xg
        f1 = jnp.maximum(
            jnp.dot(xg, wf1_ref[...], preferred_element_type=jnp.float32)
            + bf1_ref[...], 0.0)
        z_ref[...] = jnp.dot(
            f1, wf2_ref[...], preferred_element_type=jnp.float32) + bf2_ref[...]

    return pl.pallas_call(
        body,
        out_shape=[
            jax.ShapeDtypeStruct((_B, wg2.shape[1]), jnp.float32),
            jax.ShapeDtypeStruct((_B, wf2.shape[1]), jnp.float32),
        ],
    )(pmax, psum, pcnt, wg1a, wg1b, bg1, wg2, bg2, wf1, bf1, wf2, bf2)


def _split(xp, fp):
    return tuple(xp[:, i * 48:(i + 1) * 48] for i in range(fp // 48))


def _branch(x, src, dst, seg, w1, b1, w2, b2, fp):
    n, f = x.shape
    xp = jnp.pad(x, ((0, _NPAD - n), (0, fp - f)))
    w1p = jnp.pad(w1.T, ((0, fp - f), (0, fp - f)))
    b1p = jnp.pad(b1, (0, fp - f))[None, :]
    w2p = jnp.pad(w2.T, ((0, fp - f), (0, 0)))
    b2p = b2[None, :]
    segp = jnp.concatenate(
        [seg, jnp.full((_NPAD - n,), _B, seg.dtype)]).astype(jnp.float32)
    bcol = segp[:, None]
    bnext = jnp.concatenate(
        [segp[1:], jnp.full((1,), 2.0 * _B, jnp.float32)])[:, None]

    agg1 = jnp.concatenate(
        _sc_segment_sum(_split(xp, fp), src, dst), axis=1)
    h1 = _tc_gin(xp, agg1, w1p, b1p)
    agg2 = jnp.concatenate(
        _sc_segment_sum(_split(h1, fp), src, dst), axis=1)
    return _tc_gin2_pool(h1, agg2, w2p, b2p, bcol, bnext)


def kernel(data, x, edge_index, batch, a, edge, c,
           W1, b1, W2, b2, W3, b3, W4, b4,
           Wg1, bg1, Wg2, bg2, Wh1, bh1, Wh2, bh2,
           Wf1, bf1, Wf2, bf2, Wk1, bk1, Wk2, bk2):
    f2d = W2.shape[0]
    f2p = W4.shape[0]

    pmax_d, psum_d, pcnt_d = _branch(
        x, edge_index[0], edge_index[1], batch, W1, b1, W2, b2, 96)
    pmax_p, psum_p, pcnt_p = _branch(
        a, edge[0], edge[1], c, W3, b3, W4, b4, 48)

    wg1t = Wg1.T
    xg, z = _tc_head(pmax_d, psum_d, pcnt_d,
                     wg1t[:f2d], wg1t[f2d:], bg1[None, :],
                     Wg2.T, bg2[None, :], Wf1.T, bf1[None, :],
                     Wf2.T, bf2[None, :])
    wh1t = Wh1.T
    xg1, z1 = _tc_head(pmax_p, psum_p, pcnt_p,
                       wh1t[:f2p], wh1t[f2p:], bh1[None, :],
                       Wh2.T, bh2[None, :], Wk1.T, bk1[None, :],
                       Wk2.T, bk2[None, :])
    return (z, xg, xg1, z1)


# speedup vs baseline: 1.0543x; 1.0543x over previous
import functools

import jax
import jax.numpy as jnp
from jax import lax
from jax.experimental import pallas as pl
from jax.experimental.pallas import tpu as pltpu
from jax.experimental.pallas import tpu_sc as plsc

_N = 50000
_B = 256
_E = 800000
_NPAD = 50176
_NTILES = 16
_C = 2000
_G = 400
_ZR = 98
_GB = 512


def _sc_segment_sum(x_halves, src, dst):
    npad, fp = x_halves[0].shape
    n_halves = len(x_halves)
    rpr = npad // 2
    tile_rows = rpr // _NTILES
    e_per_tile = _E // _NTILES
    n_chunks = e_per_tile // _C
    nb = _C // _G

    mesh = plsc.VectorSubcoreMesh(core_axis_name="c", subcore_axis_name="s")

    @functools.partial(
        pl.kernel,
        out_type=[jax.ShapeDtypeStruct((npad, fp), jnp.float32)
                  for _ in range(n_halves)],
        mesh=mesh,
        compiler_params=pltpu.CompilerParams(use_tc_tiling_on_sc=False),
        scratch_types=[
            pltpu.VMEM((_C,), jnp.int32),
            pltpu.VMEM((_C,), jnp.int32),
            pltpu.VMEM((2, 1, _G), jnp.int32),
            pltpu.VMEM((2, 1, _G), jnp.int32),
            pltpu.VMEM((2, _G, fp), jnp.float32),
            pltpu.VMEM((_ZR, fp), jnp.float32),
            pltpu.VMEM_SHARED((rpr + _GB, fp), jnp.float32),
            pltpu.SemaphoreType.DMA,
            pltpu.SemaphoreType.DMA,
            pltpu.SemaphoreType.DMA,
            pltpu.SemaphoreType.DMA,
        ],
    )
    def kern(*refs):
        x_hbms = refs[:n_halves]
        src_hbm, dst_hbm = refs[n_halves], refs[n_halves + 1]
        out_hbms = refs[n_halves + 2:2 * n_halves + 2]
        (dstb, srcb, sidx, sgix, stag, zbuf, acc,
         gsem0, gsem1, ssem0, ssem1) = refs[2 * n_halves + 2:]
        gsems, ssems = (gsem0, gsem1), (ssem0, ssem1)
        cid = lax.axis_index("c")
        sid = lax.axis_index("s")
        lo = cid * rpr

        def zrow(i, _):
            def zcol(j, _):
                zbuf[i, pl.ds(j * 16, 16)] = jnp.zeros((16,), jnp.float32)
                return 0
            return lax.fori_loop(0, fp // 16, zcol, 0)
        lax.fori_loop(0, _ZR, zrow, 0)

        for h in range(n_halves):
            x_hbm, out_hbm = x_hbms[h], out_hbms[h]
            for zi in range(tile_rows // _ZR):
                pltpu.sync_copy(
                    zbuf, acc.at[pl.ds(sid * tile_rows + zi * _ZR, _ZR)])
            plsc.subcore_barrier()

            e0 = sid * e_per_tile

            def chunk_body(ci, _):
                base = e0 + ci * _C
                pltpu.sync_copy(dst_hbm.at[pl.ds(base, _C)], dstb)
                pltpu.sync_copy(src_hbm.at[pl.ds(base, _C)], srcb)

                def mapf_mk(bi, p):
                    def mapf(j, _):
                        d = dstb[pl.ds(bi * _G + j * 16, 16)]
                        s = srcb[pl.ds(bi * _G + j * 16, 16)]
                        inr = (d >= lo) & (d < lo + rpr)
                        loc = jnp.where(inr, d - lo,
                                        rpr + (d & (_GB - 1)))
                        sidx[p, 0, pl.ds(j * 16, 16)] = loc
                        sgix[p, 0, pl.ds(j * 16, 16)] = s
                        return 0
                    return mapf

                gd = [None, None]
                sd = [None, None]
                for b in range(nb):
                    p = b % 2
                    if b >= 2:
                        sd[p].wait()
                    lax.fori_loop(0, _G // 16, mapf_mk(b, p), 0)
                    gd[p] = pltpu.async_copy(
                        x_hbm.at[sgix.at[p, 0]], stag.at[p], gsems[p])
                    q = 1 - p
                    if b >= 1:
                        gd[q].wait()
                        sd[q] = pltpu.async_copy(
                            stag.at[q], acc.at[sidx.at[q, 0]], ssems[q],
                            add=True)
                p = (nb - 1) % 2
                gd[p].wait()
                sd[p] = pltpu.async_copy(
                    stag.at[p], acc.at[sidx.at[p, 0]], ssems[p], add=True)
                sd[1 - p].wait()
                sd[p].wait()
                return 0

            lax.fori_loop(0, n_chunks, chunk_body, 0)
            plsc.subcore_barrier()
            pltpu.sync_copy(
                acc.at[pl.ds(sid * tile_rows, tile_rows)],
                out_hbm.at[pl.ds(lo + sid * tile_rows, tile_rows)])
            plsc.subcore_barrier()

    return kern(*x_halves, src, dst)


def _tc_gin(x, agg, wp, bp):
    npad, fp = x.shape
    blk = 512

    def body(x_ref, a_ref, w_ref, b_ref, o_ref):
        t = x_ref[...] + a_ref[...]
        y = jnp.dot(t, w_ref[...], preferred_element_type=jnp.float32)
        o_ref[...] = jnp.maximum(y + b_ref[...], 0.0)

    return pl.pallas_call(
        body,
        grid=(npad // blk,),
        in_specs=[
            pl.BlockSpec((blk, fp), lambda i: (i, 0)),
            pl.BlockSpec((blk, fp), lambda i: (i, 0)),
            pl.BlockSpec(wp.shape, lambda i: (0, 0)),
            pl.BlockSpec(bp.shape, lambda i: (0, 0)),
        ],
        out_specs=pl.BlockSpec((blk, fp), lambda i: (i, 0)),
        out_shape=jax.ShapeDtypeStruct((npad, fp), jnp.float32),
    )(x, agg, wp, bp)


def _tc_gin2_pool(h1, agg, wp, bp, bcol, bnext):
    npad, fp = h1.shape
    f2 = wp.shape[1]
    blk = 256
    nblk = npad // blk

    def body(h_ref, a_ref, w_ref, b_ref, bc_ref, bn_ref,
             pmax_ref, psum_ref, pcnt_ref):
        i = pl.program_id(0)

        @pl.when(i == 0)
        def _():
            pmax_ref[...] = jnp.zeros_like(pmax_ref)
            psum_ref[...] = jnp.zeros_like(psum_ref)
            pcnt_ref[...] = jnp.zeros_like(pcnt_ref)

        t = h_ref[...] + a_ref[...]
        y = jnp.dot(t, w_ref[...], preferred_element_type=jnp.float32)
        h2 = jnp.maximum(y + b_ref[...], 0.0)

        bc = bc_ref[...]
        bn = bn_ref[...]
        segs = lax.broadcasted_iota(jnp.int32, (1, _B), 1).astype(jnp.float32)
        m = (bc == segs).astype(jnp.float32)

        dn = (((0,), (0,)), ((), ()))
        psum_ref[...] += lax.dot_general(
            m, h2, dn, preferred_element_type=jnp.float32, precision=lax.Precision.HIGHEST)
        pcnt_ref[...] += lax.dot_general(
            m, jnp.ones((blk, 1), jnp.float32), dn,
            preferred_element_type=jnp.float32, precision=lax.Precision.HIGHEST)

        c = h2
        k = 1
        while k < blk:
            sb = jnp.concatenate(
                [jnp.full((k, 1), -1.0, jnp.float32), bc[:-k]], axis=0)
            cs = jnp.concatenate(
                [jnp.zeros((k, f2), jnp.float32), c[:-k]], axis=0)
            c = jnp.maximum(c, jnp.where(sb == bc, cs, 0.0))
            k *= 2
        row_last = lax.broadcasted_iota(jnp.int32, (blk, 1), 0) == (blk - 1)
        end = jnp.logical_or(bc != bn, row_last).astype(jnp.float32)
        placed = lax.dot_general(
            m * end, c, dn, preferred_element_type=jnp.float32, precision=lax.Precision.HIGHEST)
        pmax_ref[...] = jnp.maximum(pmax_ref[...], placed)

    return pl.pallas_call(
        body,
        grid=(nblk,),
        in_specs=[
            pl.BlockSpec((blk, fp), lambda i: (i, 0)),
            pl.BlockSpec((blk, fp), lambda i: (i, 0)),
            pl.BlockSpec(wp.shape, lambda i: (0, 0)),
            pl.BlockSpec(bp.shape, lambda i: (0, 0)),
            pl.BlockSpec((blk, 1), lambda i: (i, 0)),
            pl.BlockSpec((blk, 1), lambda i: (i, 0)),
        ],
        out_specs=[
            pl.BlockSpec((_B, f2), lambda i: (0, 0)),
            pl.BlockSpec((_B, f2), lambda i: (0, 0)),
            pl.BlockSpec((_B, 1), lambda i: (0, 0)),
        ],
        out_shape=[
            jax.ShapeDtypeStruct((_B, f2), jnp.float32),
            jax.ShapeDtypeStruct((_B, f2), jnp.float32),
            jax.ShapeDtypeStruct((_B, 1), jnp.float32),
        ],
    )(h1, agg, wp, bp, bcol, bnext)


def _tc_head(pmax, psum, pcnt, wg1a, wg1b, bg1, wg2, bg2, wf1, bf1, wf2, bf2):
    def body(pmax_ref, psum_ref, pcnt_ref, wg1a_ref, wg1b_ref, bg1_ref,
             wg2_ref, bg2_ref, wf1_ref, bf1_ref, wf2_ref, bf2_ref,
             xg_ref, z_ref):
        den = jnp.maximum(pcnt_ref[...], 1.0)
        mean = psum_ref[...] / den
        g1 = (jnp.dot(pmax_ref[...], wg1a_ref[...],
                      preferred_element_type=jnp.float32)
              + jnp.dot(mean, wg1b_ref[...],
                        preferred_element_type=jnp.float32)
              + bg1_ref[...])
        g1 = jnp.maximum(g1, 0.0)
        xg = jnp.dot(g1, wg2_ref[...],
                     preferred_element_type=jnp.float32) + bg2_ref[...]
        xg_ref[...] = xg
        f1 = jnp.maximum(
            jnp.dot(xg, wf1_ref[...], preferred_element_type=jnp.float32)
            + bf1_ref[...], 0.0)
        z_ref[...] = jnp.dot(
            f1, wf2_ref[...], preferred_element_type=jnp.float32) + bf2_ref[...]

    return pl.pallas_call(
        body,
        out_shape=[
            jax.ShapeDtypeStruct((_B, wg2.shape[1]), jnp.float32),
            jax.ShapeDtypeStruct((_B, wf2.shape[1]), jnp.float32),
        ],
    )(pmax, psum, pcnt, wg1a, wg1b, bg1, wg2, bg2, wf1, bf1, wf2, bf2)


def _split(xp, fp):
    return tuple(xp[:, i * 48:(i + 1) * 48] for i in range(fp // 48))


def _branch_pre(x, src, dst, seg, w1, b1, w2, b2, fp):
    n, f = x.shape
    xp = jnp.pad(x, ((0, _NPAD - n), (0, fp - f)))
    w1p = jnp.pad(w1.T, ((0, fp - f), (0, fp - f)))
    b1p = jnp.pad(b1, (0, fp - f))[None, :]
    w2p = jnp.pad(w2.T, ((0, fp - f), (0, 0)))
    b2p = b2[None, :]
    segp = jnp.concatenate(
        [seg, jnp.full((_NPAD - n,), _B, seg.dtype)]).astype(jnp.float32)
    bcol = segp[:, None]
    bnext = jnp.concatenate(
        [segp[1:], jnp.full((1,), 2.0 * _B, jnp.float32)])[:, None]
    return xp, w1p, b1p, w2p, b2p, bcol, bnext


def kernel(data, x, edge_index, batch, a, edge, c,
           W1, b1, W2, b2, W3, b3, W4, b4,
           Wg1, bg1, Wg2, bg2, Wh1, bh1, Wh2, bh2,
           Wf1, bf1, Wf2, bf2, Wk1, bk1, Wk2, bk2):
    f2d = W2.shape[0]
    f2p = W4.shape[0]

    dxp, dw1, db1, dw2, db2, dbc, dbn = _branch_pre(
        x, edge_index[0], edge_index[1], batch, W1, b1, W2, b2, 96)
    pxp, pw1, pb1, pw2, pb2, pbc, pbn = _branch_pre(
        a, edge[0], edge[1], c, W3, b3, W4, b4, 48)

    dsrc, ddst = edge_index[0], edge_index[1]
    psrc, pdst = edge[0], edge[1]

    dagg1 = jnp.concatenate(
        _sc_segment_sum(_split(dxp, 96), dsrc, ddst), axis=1)
    pagg1 = jnp.concatenate(
        _sc_segment_sum(_split(pxp, 48), psrc, pdst), axis=1)
    dh1 = _tc_gin(dxp, dagg1, dw1, db1)
    ph1 = _tc_gin(pxp, pagg1, pw1, pb1)
    dagg2 = jnp.concatenate(
        _sc_segment_sum(_split(dh1, 96), dsrc, ddst), axis=1)
    pagg2 = jnp.concatenate(
        _sc_segment_sum(_split(ph1, 48), psrc, pdst), axis=1)
    pmax_d, psum_d, pcnt_d = _tc_gin2_pool(dh1, dagg2, dw2, db2, dbc, dbn)
    pmax_p, psum_p, pcnt_p = _tc_gin2_pool(ph1, pagg2, pw2, pb2, pbc, pbn)

    wg1t = Wg1.T
    xg, z = _tc_head(pmax_d, psum_d, pcnt_d,
                     wg1t[:f2d], wg1t[f2d:], bg1[None, :],
                     Wg2.T, bg2[None, :], Wf1.T, bf1[None, :],
                     Wf2.T, bf2[None, :])
    wh1t = Wh1.T
    xg1, z1 = _tc_head(pmax_p, psum_p, pcnt_p,
                       wh1t[:f2p], wh1t[f2p:], bh1[None, :],
                       Wh2.T, bh2[None, :], Wk1.T, bk1[None, :],
                       Wk2.T, bk2[None, :])
    return (z, xg, xg1, z1)
